# baseline (speedup 1.0000x reference)
import functools

import jax
import jax.numpy as jnp
from jax.experimental import pallas as pl
from jax.experimental.pallas import tpu as pltpu

NUM_EXPERTS = 8
TOP_K = 2
DIM = 240
BT = 512


def _moe_block(x_ref, gw_ref, gb_ref, ew_ref, eb_ref, o_ref):
    xb = x_ref[0]

    logits = jax.lax.dot_general(
        xb, gw_ref[...], (((1,), (1,)), ((), ())),
        preferred_element_type=jnp.float32,
    ) + gb_ref[...]

    idx = jax.lax.broadcasted_iota(jnp.int32, (BT, NUM_EXPERTS), 1)
    m1 = jnp.max(logits, axis=1, keepdims=True)
    i1 = jnp.min(jnp.where(logits == m1, idx, NUM_EXPERTS), axis=1, keepdims=True)
    masked = jnp.where(idx == i1, -jnp.inf, logits)
    m2 = jnp.max(masked, axis=1, keepdims=True)
    i2 = jnp.min(jnp.where(masked == m2, idx, NUM_EXPERTS), axis=1, keepdims=True)
    e2 = jnp.exp(m2 - m1)
    denom = 1.0 + e2
    w1 = 1.0 / denom
    w2 = e2 / denom
    wfull = (jnp.where(idx == i1, w1, 0.0) + jnp.where(idx == i2, w2, 0.0))

    xb16 = xb.astype(jnp.bfloat16)
    acc = jax.lax.dot_general(
        wfull, eb_ref[...], (((1,), (0,)), ((), ())),
        preferred_element_type=jnp.float32,
    )
    for e in range(NUM_EXPERTS):
        we = wfull[:, e:e + 1]
        ye = jax.lax.dot_general(
            xb16, ew_ref[e], (((1,), (1,)), ((), ())),
            preferred_element_type=jnp.float32,
        )
        acc = acc + we * ye
    o_ref[0] = acc


@jax.jit
def kernel(x, gate_w, gate_b, expert_w, expert_b):
    b, s, d = x.shape
    gb2 = gate_b.reshape(1, NUM_EXPERTS)
    ew16 = expert_w.astype(jnp.bfloat16)

    return pl.pallas_call(
        _moe_block,
        grid=(b,),
        in_specs=[
            pl.BlockSpec((1, BT, d), lambda i: (i, 0, 0)),
            pl.BlockSpec((NUM_EXPERTS, d), lambda i: (0, 0)),
            pl.BlockSpec((1, NUM_EXPERTS), lambda i: (0, 0)),
            pl.BlockSpec((NUM_EXPERTS, d, d), lambda i: (0, 0, 0)),
            pl.BlockSpec((NUM_EXPERTS, d), lambda i: (0, 0)),
        ],
        out_specs=pl.BlockSpec((1, BT, d), lambda i: (i, 0, 0)),
        out_shape=jax.ShapeDtypeStruct((b, s, d), jnp.float32),
        compiler_params=pltpu.CompilerParams(
            dimension_semantics=("arbitrary",),
        ),
    )(x, gate_w, gb2, ew16, expert_b)


# speedup vs baseline: 1.3809x; 1.3809x over previous
import functools

import jax
import jax.numpy as jnp
from jax.experimental import pallas as pl
from jax.experimental.pallas import tpu as pltpu

NUM_EXPERTS = 8
TOP_K = 2
DIM = 240
BR = 4
BT = BR * 512


def _moe_block(x_ref, gw_ref, gb_ref, ew_ref, eb_ref, o_ref):
    xb = x_ref[...].reshape(BT, DIM)

    logits = jax.lax.dot_general(
        xb, gw_ref[...], (((1,), (1,)), ((), ())),
        preferred_element_type=jnp.float32,
    ) + gb_ref[...]

    idx = jax.lax.broadcasted_iota(jnp.int32, (BT, NUM_EXPERTS), 1)
    m1 = jnp.max(logits, axis=1, keepdims=True)
    i1 = jnp.min(jnp.where(logits == m1, idx, NUM_EXPERTS), axis=1, keepdims=True)
    masked = jnp.where(idx == i1, -jnp.inf, logits)
    m2 = jnp.max(masked, axis=1, keepdims=True)
    i2 = jnp.min(jnp.where(masked == m2, idx, NUM_EXPERTS), axis=1, keepdims=True)
    e2 = jnp.exp(m2 - m1)
    denom = 1.0 + e2
    w1 = 1.0 / denom
    w2 = e2 / denom
    wfull = (jnp.where(idx == i1, w1, 0.0) + jnp.where(idx == i2, w2, 0.0))

    xb16 = xb.astype(jnp.bfloat16)
    acc = jax.lax.dot_general(
        wfull, eb_ref[...], (((1,), (0,)), ((), ())),
        preferred_element_type=jnp.float32,
    )
    for e in range(NUM_EXPERTS):
        we = wfull[:, e:e + 1]
        ye = jax.lax.dot_general(
            xb16, ew_ref[e], (((1,), (1,)), ((), ())),
            preferred_element_type=jnp.float32,
        )
        acc = acc + we * ye
    o_ref[...] = acc.reshape(BR, 512, DIM)


@jax.jit
def kernel(x, gate_w, gate_b, expert_w, expert_b):
    b, s, d = x.shape
    gb2 = gate_b.reshape(1, NUM_EXPERTS)
    ew16 = expert_w.astype(jnp.bfloat16)

    return pl.pallas_call(
        _moe_block,
        grid=(b // BR,),
        in_specs=[
            pl.BlockSpec((BR, 512, d), lambda i: (i, 0, 0)),
            pl.BlockSpec((NUM_EXPERTS, d), lambda i: (0, 0)),
            pl.BlockSpec((1, NUM_EXPERTS), lambda i: (0, 0)),
            pl.BlockSpec((NUM_EXPERTS, d, d), lambda i: (0, 0, 0)),
            pl.BlockSpec((NUM_EXPERTS, d), lambda i: (0, 0)),
        ],
        out_specs=pl.BlockSpec((BR, 512, d), lambda i: (i, 0, 0)),
        out_shape=jax.ShapeDtypeStruct((b, s, d), jnp.float32),
        compiler_params=pltpu.CompilerParams(
            dimension_semantics=("arbitrary",),
        ),
    )(x, gate_w, gb2, ew16, expert_b)
